# baseline (speedup 1.0000x reference)
import functools

import jax
import jax.numpy as jnp
from jax import lax
from jax.experimental import pallas as pl
from jax.experimental.pallas import tpu as pltpu
from jax.experimental.pallas import tpu_sc as plsc

NC = 2
NS = 16
NW = NC * NS
L = 16

B = 2
CH = 128
C = 44
H = W = D = 40
BLK = W * D
ROWS = B * CH * H
CH_PER_W = CH // NW


def _body(tbl_hbm, out_hbm, tbl_v, blk_v, rowbuf_v, sem):
    cid = lax.axis_index("c")
    sid = lax.axis_index("s")
    wid = sid * NC + cid

    pltpu.sync_copy(tbl_hbm, tbl_v)
    lanes = lax.broadcasted_iota(jnp.int32, (L,), 0)

    for q in range(CH_PER_W):
        ch = wid + NW * q
        t = ch // C
        c = ch - t * C
        c_vec = jnp.full((L,), c, jnp.int32)

        @pl.when(t == 0)
        def _():
            def build(g, carry):
                pos = g * L + lanes
                blk_v[pl.ds(g * L, L)] = plsc.load_gather(tbl_v, [pos // D, c_vec])
                return carry

            lax.fori_loop(0, BLK // L, build, 0)

        @pl.when(t == 2)
        def _():
            def build(g, carry):
                pos = g * L + lanes
                blk_v[pl.ds(g * L, L)] = plsc.load_gather(tbl_v, [100 + pos % D, c_vec])
                return carry

            lax.fori_loop(0, BLK // L, build, 0)

        @pl.when(t != 1)
        def _():
            def fire(i, carry):
                b = i // H
                h = i - b * H
                r = b * (CH * H) + ch * H + h
                pltpu.make_async_copy(blk_v, out_hbm.at[r], sem).start()
                return carry

            lax.fori_loop(0, B * H, fire, 0)
            pltpu.make_async_copy(out_hbm.at[pl.ds(0, H)], rowbuf_v, sem).wait()
            pltpu.make_async_copy(out_hbm.at[pl.ds(0, H)], rowbuf_v, sem).wait()

        @pl.when(t == 1)
        def _():
            def hbody(h, carry):
                h_vec = jnp.full((L,), 50 + h, jnp.int32)
                cval = plsc.load_gather(tbl_v, [h_vec, c_vec])

                def fill(g, inner):
                    rowbuf_v[h, pl.ds(g * L, L)] = cval
                    return inner

                lax.fori_loop(0, BLK // L, fill, 0)
                return carry

            lax.fori_loop(0, H, hbody, 0)
            pltpu.make_async_copy(rowbuf_v, out_hbm.at[pl.ds(ch * H, H)], sem).start()
            pltpu.make_async_copy(rowbuf_v, out_hbm.at[pl.ds(CH * H + ch * H, H)], sem).start()
            pltpu.make_async_copy(out_hbm.at[pl.ds(0, H)], rowbuf_v.at[pl.ds(0, H)], sem).wait()
            pltpu.make_async_copy(out_hbm.at[pl.ds(0, H)], rowbuf_v.at[pl.ds(0, H)], sem).wait()


@jax.jit
def _pos_embed(tbl):
    mesh = plsc.VectorSubcoreMesh(core_axis_name="c", subcore_axis_name="s")
    f = pl.kernel(
        _body,
        out_type=jax.ShapeDtypeStruct((ROWS, BLK), jnp.float32),
        mesh=mesh,
        compiler_params=pltpu.CompilerParams(needs_layout_passes=False),
        scratch_types=[
            pltpu.VMEM((150, C), jnp.float32),
            pltpu.VMEM((BLK,), jnp.float32),
            pltpu.VMEM((H, BLK), jnp.float32),
            pltpu.SemaphoreType.DMA,
        ],
    )
    return f(tbl)


def kernel(x, row_embed, col_embed, depth_embed):
    tbl = jnp.concatenate([col_embed, row_embed, depth_embed], axis=0)
    out = _pos_embed(tbl)
    return out.reshape(B, CH, H, W, D)


# speedup vs baseline: 6.6958x; 6.6958x over previous
import jax
import jax.numpy as jnp
from jax import lax
from jax.experimental import pallas as pl
from jax.experimental.pallas import tpu as pltpu
from jax.experimental.pallas import tpu_sc as plsc

NC = 2
NS = 16
NW = NC * NS
L = 16

B = 2
CH = 128
C = 44
H = W = D = 40
WG = 5
GROUPS = H * (W // WG)
G_PER_W = GROUPS // NW
GROWS = WG * D
OUT_ROWS = B * H * W * D

PAT_OFFS = (0, 16, 32, 48, 64, 72)
DEP_OFFS = (88, 104, 112)


def _body(tbl_hbm, out_hbm, tbl_v, buf0, buf1, sem0, sem1):
    cid = lax.axis_index("c")
    sid = lax.axis_index("s")
    wid = sid * NC + cid

    pltpu.sync_copy(tbl_hbm, tbl_v)
    lanes = lax.broadcasted_iota(jnp.int32, (L,), 0)

    bufs = (buf0, buf1)
    sems = (sem0, sem1)

    for buf in bufs:
        def dinit(r, carry):
            d_row = jnp.full((L,), 100 + lax.rem(r, D), jnp.int32)
            for a in DEP_OFFS:
                cols = (a - 88) + lanes
                buf[r, pl.ds(a, L)] = plsc.load_gather(tbl_v, [d_row, cols])
            return carry

        lax.fori_loop(0, GROWS, dinit, 0)

    for i in range(G_PER_W):
        buf = bufs[i % 2]
        sem = sems[i % 2]
        if i >= 2:
            pltpu.make_async_copy(out_hbm.at[pl.ds(0, GROWS)], buf, sem).wait()
            pltpu.make_async_copy(out_hbm.at[pl.ds(0, GROWS)], buf, sem).wait()

        gg = wid * G_PER_W + i
        h = gg // (W // WG)
        w0 = (gg - h * (W // WG)) * WG

        def sub_body(sub, carry):
            w = w0 + sub
            vs = []
            for a in PAT_OFFS:
                p = a + lanes
                in_col = p < C
                ridx = jnp.where(in_col, jnp.full((L,), w, jnp.int32),
                                 jnp.full((L,), 50 + h, jnp.int32))
                cidx = jnp.where(in_col, p, p - C)
                vs.append(plsc.load_gather(tbl_v, [ridx, cidx]))

            def fill(rr, inner):
                r = sub * D + rr
                for a, v in zip(PAT_OFFS, vs):
                    buf[r, pl.ds(a, L)] = v
                return inner

            lax.fori_loop(0, D, fill, 0)
            return carry

        lax.fori_loop(0, WG, sub_body, 0)

        base = (h * W + w0) * D
        pltpu.make_async_copy(buf, out_hbm.at[pl.ds(base, GROWS)], sem).start()
        pltpu.make_async_copy(buf, out_hbm.at[pl.ds(H * W * D + base, GROWS)], sem).start()

    for i in (G_PER_W - 2, G_PER_W - 1):
        buf = bufs[i % 2]
        sem = sems[i % 2]
        pltpu.make_async_copy(out_hbm.at[pl.ds(0, GROWS)], buf, sem).wait()
        pltpu.make_async_copy(out_hbm.at[pl.ds(0, GROWS)], buf, sem).wait()


@jax.jit
def _pos_embed(tbl):
    mesh = plsc.VectorSubcoreMesh(core_axis_name="c", subcore_axis_name="s")
    f = pl.kernel(
        _body,
        out_type=jax.ShapeDtypeStruct((OUT_ROWS, CH), jnp.float32),
        mesh=mesh,
        compiler_params=pltpu.CompilerParams(needs_layout_passes=False),
        scratch_types=[
            pltpu.VMEM((150, C), jnp.float32),
            pltpu.VMEM((GROWS, CH), jnp.float32),
            pltpu.VMEM((GROWS, CH), jnp.float32),
            pltpu.SemaphoreType.DMA,
            pltpu.SemaphoreType.DMA,
        ],
    )
    out = f(tbl)
    return out.reshape(B, H, W, D, CH).transpose(0, 4, 1, 2, 3)


def kernel(x, row_embed, col_embed, depth_embed):
    tbl = jnp.concatenate([col_embed, row_embed, depth_embed], axis=0)
    return _pos_embed(tbl)
